# baseline (speedup 1.0000x reference)
import functools

import jax
import jax.numpy as jnp
from jax import lax
from jax.experimental import pallas as pl
from jax.experimental.pallas import tpu as pltpu
from jax.experimental.pallas import tpu_sc as plsc

_SMOOTH = 1e-05
_NC = 2
_NS = 16
_NW = _NC * _NS
_CPAD = 32


def _sc_body(B, C, N, CH, p_hbm, t_hbm, bins_hbm, sq_hbm,
             pbuf, tbuf, bins, sqbuf):
    slc = N // _NW
    wid = lax.axis_index("s") * _NC + lax.axis_index("c")
    n0 = wid * slc
    ones = jnp.ones((16,), jnp.float32)
    cols0 = lax.iota(jnp.int32, 16)

    zero16 = jnp.zeros((16,), jnp.float32)
    for q in range(2):
        for b in range(B):
            for j in range(_CPAD // 16):
                bins[q, b, pl.ds(j * 16, 16)] = zero16

    for b in range(B):
        pltpu.sync_copy(t_hbm.at[b, pl.ds(n0, slc)], tbuf)
        sq = [jnp.zeros((16,), jnp.float32) for _ in range(C)]
        for k in range(slc // CH):
            pltpu.sync_copy(
                p_hbm.at[pl.ds(b * C, C), pl.ds(n0 + k * CH, CH)], pbuf)

            def group(g, sqc):
                tv = tbuf[pl.ds(k * CH + g * 16, 16)]
                cols = cols0 + g * 16
                pv = plsc.load_gather(pbuf, [tv, cols])
                bvec = jnp.full((16,), b, jnp.int32)
                qi = jnp.full((16,), 0, jnp.int32)
                qc = jnp.full((16,), 1, jnp.int32)
                plsc.addupdate_scatter(bins, [qi, bvec, tv], pv)
                plsc.addupdate_scatter(bins, [qc, bvec, tv], ones)
                out = []
                for c in range(C):
                    pc = pbuf[c, pl.ds(g * 16, 16)]
                    out.append(sqc[c] + pc * pc)
                return tuple(out)

            sq = lax.fori_loop(0, CH // 16, group, tuple(sq))
        for c in range(C):
            sqbuf[b, c, :] = sq[c]

    pltpu.sync_copy(bins, bins_hbm.at[wid])
    pltpu.sync_copy(sqbuf, sq_hbm.at[wid])


def _combine_body(bins_ref, sq_ref, out_ref):
    s = jnp.sum(bins_ref[...], axis=0)
    inter = s[0]
    cnt = s[1]
    B, CP = inter.shape
    C = sq_ref.shape[2]
    sqs = jnp.sum(sq_ref[...], axis=(0, 3))
    sqp = jnp.concatenate(
        [sqs, jnp.zeros((B, CP - C), jnp.float32)], axis=1)
    dice = (2.0 * inter + _SMOOTH) / (sqp + cnt + _SMOOTH)
    valid = jax.lax.broadcasted_iota(jnp.int32, dice.shape, 1) < C
    dsum = jnp.sum(jnp.where(valid, dice, 0.0))
    out_ref[...] = jnp.full((1, 1), 1.0 - dsum / (B * C), jnp.float32)


@jax.jit
def _dice_loss_sc(predict, target):
    B, C, N = predict.shape
    p2 = predict.reshape(B * C, N)
    t2 = target.astype(jnp.int32).reshape(B, N)
    CH = 4096
    mesh = plsc.VectorSubcoreMesh(core_axis_name="c", subcore_axis_name="s")
    sc = pl.kernel(
        functools.partial(_sc_body, B, C, N, CH),
        out_type=(
            jax.ShapeDtypeStruct((_NW, 2, B, _CPAD), jnp.float32),
            jax.ShapeDtypeStruct((_NW, B, C, 16), jnp.float32),
        ),
        mesh=mesh,
        compiler_params=pltpu.CompilerParams(use_tc_tiling_on_sc=False, needs_layout_passes=False, skip_device_barrier=True),
        scratch_types=[
            pltpu.VMEM((C, CH), jnp.float32),
            pltpu.VMEM((N // _NW,), jnp.int32),
            pltpu.VMEM((2, B, _CPAD), jnp.float32),
            pltpu.VMEM((B, C, 16), jnp.float32),
        ],
    )
    bins, sqv = sc(p2, t2)
    out = pl.pallas_call(
        _combine_body,
        out_shape=jax.ShapeDtypeStruct((1, 1), jnp.float32),
    )(bins, sqv)
    return out[0, 0]


def kernel(predict, target):
    return _dice_loss_sc(predict, target)


# speedup vs baseline: 3.5859x; 3.5859x over previous
import functools

import jax
import jax.numpy as jnp
from jax import lax
from jax.experimental import pallas as pl
from jax.experimental.pallas import tpu as pltpu
from jax.experimental.pallas import tpu_sc as plsc

_SMOOTH = 1e-05
_NC = 2
_NS = 16
_NW = _NC * _NS
_CPAD = 32
_NL = 128
_KT = 8


def _sc_body(B, C, NT, p_hbm, t_hbm, bins_hbm, sq_hbm, pbuf, tbuf, bins, sqbuf):
    nt_per = NT // _NW
    npix = nt_per * _NL
    wid = lax.axis_index("s") * _NC + lax.axis_index("c")
    nt0 = wid * nt_per
    onesf = jnp.ones((16,), jnp.float32)
    zeros16i = jnp.zeros((16,), jnp.int32)
    ones16i = jnp.ones((16,), jnp.int32)
    cols0 = lax.iota(jnp.int32, 16)
    zero16 = jnp.zeros((16,), jnp.float32)

    for q in range(2):
        for b in range(B):
            for j in range(_CPAD // 16):
                bins[q, b, pl.ds(j * 16, 16)] = zero16
    for b in range(B):
        for c in range(C):
            sqbuf[b, c, :] = zero16

    for b in range(B):
        pltpu.sync_copy(t_hbm.at[b, pl.ds(nt0 * _NL, npix)], tbuf.at[b])

    for k in range(nt_per // _KT):
        pltpu.sync_copy(p_hbm.at[:, pl.ds(nt0 + k * _KT, _KT)], pbuf)
        for b in range(B):
            bv16 = jnp.full((16,), b, jnp.int32)

            def group(g, _, b=b, bv16=bv16, k=k):
                jj = (g % 8) * 16
                dnt = g // 8
                tv = tbuf[b, pl.ds(k * (_KT * _NL) + g * 16, 16)]
                nlv = cols0 + jj
                dntv = jnp.full((16,), dnt, jnp.int32)
                pv = plsc.load_gather(pbuf, [tv, dntv, bv16, nlv])
                plsc.addupdate_scatter(bins, [zeros16i, bv16, tv], pv)
                plsc.addupdate_scatter(bins, [ones16i, bv16, tv], onesf)
                for c in range(C):
                    pc = pbuf[c, dnt, b, pl.ds(jj, 16)]
                    plsc.addupdate(sqbuf.at[b, c], pc * pc)
                return 0

            lax.fori_loop(0, _KT * 8, group, 0)

    pltpu.sync_copy(bins, bins_hbm.at[wid])
    pltpu.sync_copy(sqbuf, sq_hbm.at[wid])


def _combine_body(bins_ref, sq_ref, out_ref):
    s = jnp.sum(bins_ref[...], axis=0)
    inter = s[0]
    cnt = s[1]
    B, CP = inter.shape
    C = sq_ref.shape[2]
    sqs = jnp.sum(sq_ref[...], axis=(0, 3))
    sqp = jnp.concatenate(
        [sqs, jnp.zeros((B, CP - C), jnp.float32)], axis=1)
    dice = (2.0 * inter + _SMOOTH) / (sqp + cnt + _SMOOTH)
    valid = jax.lax.broadcasted_iota(jnp.int32, dice.shape, 1) < C
    dsum = jnp.sum(jnp.where(valid, dice, 0.0))
    out_ref[...] = jnp.full((1, 1), 1.0 - dsum / (B * C), jnp.float32)


@jax.jit
def _dice_loss_sc(predict, target):
    B, C, N = predict.shape
    NT = N // _NL
    pt = predict.reshape(B, C, NT, _NL).transpose(1, 2, 0, 3)
    t2 = target.astype(jnp.int32).reshape(B, N)
    mesh = plsc.VectorSubcoreMesh(core_axis_name="c", subcore_axis_name="s")
    sc = pl.kernel(
        functools.partial(_sc_body, B, C, NT),
        out_type=(
            jax.ShapeDtypeStruct((_NW, 2, B, _CPAD), jnp.float32),
            jax.ShapeDtypeStruct((_NW, B, C, 16), jnp.float32),
        ),
        mesh=mesh,
        compiler_params=pltpu.CompilerParams(
            use_tc_tiling_on_sc=False, needs_layout_passes=False,
            skip_device_barrier=True),
        scratch_types=[
            pltpu.VMEM((C, _KT, B, _NL), jnp.float32),
            pltpu.VMEM((B, (N // _NW)), jnp.int32),
            pltpu.VMEM((2, B, _CPAD), jnp.float32),
            pltpu.VMEM((B, C, 16), jnp.float32),
        ],
    )
    bins, sqv = sc(pt, t2)
    out = pl.pallas_call(
        _combine_body,
        out_shape=jax.ShapeDtypeStruct((1, 1), jnp.float32),
    )(bins, sqv)
    return out[0, 0]


def kernel(predict, target):
    return _dice_loss_sc(predict, target)


# speedup vs baseline: 12.1202x; 3.3800x over previous
import functools

import jax
import jax.numpy as jnp
from jax import lax
from jax.experimental import pallas as pl
from jax.experimental.pallas import tpu as pltpu
from jax.experimental.pallas import tpu_sc as plsc

_SMOOTH = 1e-05
_NC = 2
_NS = 16
_NW = _NC * _NS
_CPAD = 32
_NL = 128
_KT = 4


def _sc_body(B, C, NT, p_hbm, t_hbm, bins_hbm, sq_hbm, pbuf, tbuf, bins,
             sqbuf, sem0, sem1):
    nt_per = NT // _NW
    npix = nt_per * _NL
    wid = lax.axis_index("s") * _NC + lax.axis_index("c")
    nt0 = wid * nt_per
    onesf = jnp.ones((16,), jnp.float32)
    zeros16i = jnp.zeros((16,), jnp.int32)
    ones16i = jnp.ones((16,), jnp.int32)
    cols0 = lax.iota(jnp.int32, 16)
    zero16 = jnp.zeros((16,), jnp.float32)

    for q in range(2):
        for b in range(B):
            for j in range(_CPAD // 16):
                bins[q, b, pl.ds(j * 16, 16)] = zero16
    for b in range(B):
        for c in range(C):
            sqbuf[b, c, :] = zero16

    for b in range(B):
        pltpu.sync_copy(t_hbm.at[b, pl.ds(nt0 * _NL, npix)], tbuf.at[b])

    nchunks = nt_per // _KT
    waits = [None, None]
    waits[0] = pltpu.async_copy(
        p_hbm.at[:, pl.ds(nt0, _KT)], pbuf.at[0], sem0)
    for k in range(nchunks):
        cur = k % 2
        if k + 1 < nchunks:
            nxt = 1 - cur
            nsem = sem0 if nxt == 0 else sem1
            waits[nxt] = pltpu.async_copy(
                p_hbm.at[:, pl.ds(nt0 + (k + 1) * _KT, _KT)],
                pbuf.at[nxt], nsem)
        waits[cur].wait()
        pb = pbuf.at[cur]
        for b in range(B):
            bv16 = jnp.full((16,), b, jnp.int32)

            def group(g, sqc, b=b, bv16=bv16, k=k, pb=pb):
                jj = (g % 8) * 16
                dnt = g // 8
                tv = tbuf[b, pl.ds(k * (_KT * _NL) + g * 16, 16)]
                nlv = cols0 + jj
                dntv = jnp.full((16,), dnt, jnp.int32)
                pv = plsc.load_gather(pb, [tv, dntv, bv16, nlv])
                plsc.addupdate_scatter(bins, [zeros16i, bv16, tv], pv)
                plsc.addupdate_scatter(bins, [ones16i, bv16, tv], onesf)
                out = []
                for c in range(C):
                    pc = pb[c, dnt, b, pl.ds(jj, 16)]
                    out.append(sqc[c] + pc * pc)
                return tuple(out)

            sq = lax.fori_loop(0, _KT * 8, group,
                               tuple(zero16 for _ in range(C)))
            for c in range(C):
                plsc.addupdate(sqbuf.at[b, c], sq[c])

    pltpu.sync_copy(bins, bins_hbm.at[wid])
    pltpu.sync_copy(sqbuf, sq_hbm.at[wid])


def _combine_body(bins_ref, sq_ref, out_ref):
    s = jnp.sum(bins_ref[...], axis=0)
    inter = s[0]
    cnt = s[1]
    B, CP = inter.shape
    C = sq_ref.shape[2]
    sqs = jnp.sum(sq_ref[...], axis=(0, 3))
    sqp = jnp.concatenate(
        [sqs, jnp.zeros((B, CP - C), jnp.float32)], axis=1)
    dice = (2.0 * inter + _SMOOTH) / (sqp + cnt + _SMOOTH)
    valid = jax.lax.broadcasted_iota(jnp.int32, dice.shape, 1) < C
    dsum = jnp.sum(jnp.where(valid, dice, 0.0))
    out_ref[...] = jnp.full((1, 1), 1.0 - dsum / (B * C), jnp.float32)


@jax.jit
def _dice_loss_sc(predict, target):
    B, C, N = predict.shape
    NT = N // _NL
    pt = predict.reshape(B, C, NT, _NL).transpose(1, 2, 0, 3)
    t2 = target.astype(jnp.int32).reshape(B, N)
    mesh = plsc.VectorSubcoreMesh(core_axis_name="c", subcore_axis_name="s")
    sc = pl.kernel(
        functools.partial(_sc_body, B, C, NT),
        out_type=(
            jax.ShapeDtypeStruct((_NW, 2, B, _CPAD), jnp.float32),
            jax.ShapeDtypeStruct((_NW, B, C, 16), jnp.float32),
        ),
        mesh=mesh,
        compiler_params=pltpu.CompilerParams(
            use_tc_tiling_on_sc=False, needs_layout_passes=False,
            skip_device_barrier=True),
        scratch_types=[
            pltpu.VMEM((2, C, _KT, B, _NL), jnp.float32),
            pltpu.VMEM((B, (N // _NW)), jnp.int32),
            pltpu.VMEM((2, B, _CPAD), jnp.float32),
            pltpu.VMEM((B, C, 16), jnp.float32),
            pltpu.SemaphoreType.DMA,
            pltpu.SemaphoreType.DMA,
        ],
    )
    bins, sqv = sc(pt, t2)
    out = pl.pallas_call(
        _combine_body,
        out_shape=jax.ShapeDtypeStruct((1, 1), jnp.float32),
    )(bins, sqv)
    return out[0, 0]


def kernel(predict, target):
    return _dice_loss_sc(predict, target)
